# baseline (speedup 1.0000x reference)
import jax
import jax.numpy as jnp
from jax import lax
from jax.experimental import pallas as pl
from jax.experimental.pallas import tpu as pltpu
from jax.experimental.pallas import tpu_sc as plsc

NC = 2
NS = 16
NW = NC * NS

D = 64
L = 128
NBUF = 2


def _emb_body(idx_hbm, table_hbm, out_hbm, idx_v, rows_v, trows_v,
              gsem0, gsem1, wsem0, wsem1):
  gsems = (gsem0, gsem1)
  wsems = (wsem0, wsem1)
  wid = lax.axis_index("s") * NC + lax.axis_index("c")
  units = idx_hbm.shape[0] // NW
  row0 = wid * units

  pltpu.sync_copy(idx_hbm.at[pl.ds(row0, units)], idx_v)

  iota16 = lax.iota(jnp.int32, 16)
  rowvecs = [iota16 + 16 * k for k in range(8)]

  def fire_gather(u, b):
    pltpu.async_copy(table_hbm.at[idx_v.at[u]], rows_v.at[b], gsems[b])

  def wait_gather(u, b):
    pltpu.make_async_copy(table_hbm.at[idx_v.at[u]], rows_v.at[b],
                          gsems[b]).wait()

  def transpose(b):
    def eo_body(eo, carry):
      for sub in range(8):
        e = eo * 8 + sub
        colv = jnp.broadcast_to(e, (16,))
        for k8 in range(8):
          v = plsc.load_gather(rows_v.at[b], [rowvecs[k8], colv])
          trows_v[b, eo, sub, pl.ds(k8 * 16, 16)] = v
      return carry
    lax.fori_loop(0, 8, eo_body, 0)

  def fire_wb(u, b):
    q = row0 + u
    s = q // 32
    bt = lax.rem(q, 32)
    for eo in range(8):
      pltpu.async_copy(trows_v.at[b, eo], out_hbm.at[s, eo, bt], wsems[b])

  def wait_wb(u, b):
    q = row0 + u
    s = q // 32
    bt = lax.rem(q, 32)
    for eo in range(8):
      pltpu.make_async_copy(trows_v.at[b, eo], out_hbm.at[s, eo, bt],
                            wsems[b]).wait()

  fire_gather(0, 0)

  wait_gather(0, 0)
  fire_gather(1, 1)
  transpose(0)
  fire_wb(0, 0)

  wait_gather(1, 1)
  fire_gather(2, 0)
  transpose(1)
  fire_wb(1, 1)

  def pair_body(p, carry):
    for db in range(NBUF):
      u = 2 * p + db
      b = db
      nb = 1 - db
      wait_gather(u, b)
      fire_gather(u + 1, nb)
      wait_wb(u - 2, b)
      transpose(b)
      fire_wb(u, b)
    return carry

  lax.fori_loop(1, units // 2 - 1, pair_body, 0)

  wait_gather(units - 2, 0)
  fire_gather(units - 1, 1)
  wait_wb(units - 4, 0)
  transpose(0)
  fire_wb(units - 2, 0)

  wait_gather(units - 1, 1)
  wait_wb(units - 3, 1)
  transpose(1)
  fire_wb(units - 1, 1)

  wait_wb(units - 2, 0)
  wait_wb(units - 1, 1)


def kernel(src, seg, table):
  del seg
  B, S = src.shape
  n = B * S
  idx = src.T.reshape(n // L, L).astype(jnp.int32)
  mesh = plsc.VectorSubcoreMesh(core_axis_name="c", subcore_axis_name="s",
                                num_cores=NC, num_subcores=NS)
  run = pl.kernel(
      _emb_body,
      out_type=jax.ShapeDtypeStruct((S, D // 8, B // L, 8, L), jnp.float32),
      mesh=mesh,
      scratch_types=[
          pltpu.VMEM((n // L // NW, L), jnp.int32),
          pltpu.VMEM((NBUF, L, D), jnp.float32),
          pltpu.VMEM((NBUF, 8, 8, L), jnp.float32),
          pltpu.SemaphoreType.DMA,
          pltpu.SemaphoreType.DMA,
          pltpu.SemaphoreType.DMA,
          pltpu.SemaphoreType.DMA,
      ],
      compiler_params=pltpu.CompilerParams(use_tc_tiling_on_sc=False,
                                           needs_layout_passes=False),
  )
  out5 = run(idx, table)
  return out5.transpose((2, 4, 0, 1, 3)).reshape(B, S, D)


# speedup vs baseline: 1.2823x; 1.2823x over previous
import jax
import jax.numpy as jnp
from jax import lax
from jax.experimental import pallas as pl
from jax.experimental.pallas import tpu as pltpu
from jax.experimental.pallas import tpu_sc as plsc

NC = 2
NS = 16
NW = NC * NS

D = 64
L = 128
P = 65
NBUF = 2


def _emb_body(idx_hbm, table_hbm, out_hbm, idx_v, rows_v, rows_p, trows_v,
              gsem0, gsem1, wsem0, wsem1):
  gsems = (gsem0, gsem1)
  wsems = (wsem0, wsem1)
  wid = lax.axis_index("s") * NC + lax.axis_index("c")
  units = idx_hbm.shape[0] // NW
  row0 = wid * units

  pltpu.sync_copy(idx_hbm.at[pl.ds(row0, units)], idx_v)

  iota16 = lax.iota(jnp.int32, 16)
  rowvecs = [iota16 + 16 * k for k in range(8)]

  def fire_gather(u, b):
    pltpu.async_copy(table_hbm.at[idx_v.at[u]], rows_v.at[b], gsems[b])

  def wait_gather(u, b):
    pltpu.make_async_copy(table_hbm.at[idx_v.at[u]], rows_v.at[b],
                          gsems[b]).wait()

  def transpose(b):
    def i_body(i, carry):
      for e0 in (0, 16, 32, 48):
        rows_p[b, i, pl.ds(e0, 16)] = rows_v[b, i, pl.ds(e0, 16)]
      return carry
    lax.fori_loop(0, L, i_body, 0)

    def e_body(e, carry):
      colv = jnp.broadcast_to(e, (16,))
      for k8 in range(8):
        v = plsc.load_gather(rows_p.at[b], [rowvecs[k8], colv])
        trows_v[b, e, pl.ds(16 * k8, 16)] = v
      return carry
    lax.fori_loop(0, D, e_body, 0)

  def fire_wb(u, b):
    q = row0 + u
    s = q // 32
    bt = lax.rem(q, 32)
    for eo in range(8):
      pltpu.async_copy(trows_v.at[b].at[pl.ds(eo * 8, 8)],
                       out_hbm.at[s, eo, bt], wsems[b])

  def wait_wb(u, b):
    q = row0 + u
    s = q // 32
    bt = lax.rem(q, 32)
    for eo in range(8):
      pltpu.make_async_copy(trows_v.at[b].at[pl.ds(eo * 8, 8)],
                            out_hbm.at[s, eo, bt], wsems[b]).wait()

  fire_gather(0, 0)

  wait_gather(0, 0)
  fire_gather(1, 1)
  transpose(0)
  fire_wb(0, 0)

  wait_gather(1, 1)
  fire_gather(2, 0)
  transpose(1)
  fire_wb(1, 1)

  def pair_body(p, carry):
    for db in range(NBUF):
      u = 2 * p + db
      b = db
      nb = 1 - db
      wait_gather(u, b)
      fire_gather(u + 1, nb)
      wait_wb(u - 2, b)
      transpose(b)
      fire_wb(u, b)
    return carry

  lax.fori_loop(1, units // 2 - 1, pair_body, 0)

  wait_gather(units - 2, 0)
  fire_gather(units - 1, 1)
  wait_wb(units - 4, 0)
  transpose(0)
  fire_wb(units - 2, 0)

  wait_gather(units - 1, 1)
  wait_wb(units - 3, 1)
  transpose(1)
  fire_wb(units - 1, 1)

  wait_wb(units - 2, 0)
  wait_wb(units - 1, 1)


def kernel(src, seg, table):
  del seg
  B, S = src.shape
  n = B * S
  idx = src.T.reshape(n // L, L).astype(jnp.int32)
  table_p = jnp.pad(table, ((0, 0), (0, L - D)))
  mesh = plsc.VectorSubcoreMesh(core_axis_name="c", subcore_axis_name="s",
                                num_cores=NC, num_subcores=NS)
  run = pl.kernel(
      _emb_body,
      out_type=jax.ShapeDtypeStruct((S, D // 8, B // L, 8, L), jnp.float32),
      mesh=mesh,
      scratch_types=[
          pltpu.VMEM((n // L // NW, L), jnp.int32),
          pltpu.VMEM((NBUF, L, L), jnp.float32),
          pltpu.VMEM((NBUF, L, P), jnp.float32),
          pltpu.VMEM((NBUF, D, L), jnp.float32),
          pltpu.SemaphoreType.DMA,
          pltpu.SemaphoreType.DMA,
          pltpu.SemaphoreType.DMA,
          pltpu.SemaphoreType.DMA,
      ],
      compiler_params=pltpu.CompilerParams(use_tc_tiling_on_sc=False,
                                           needs_layout_passes=False),
  )
  out5 = run(idx, table_p)
  return out5.transpose((2, 4, 0, 1, 3)).reshape(B, S, D)


# speedup vs baseline: 2.2165x; 1.7285x over previous
import jax
import jax.numpy as jnp
from jax import lax
from jax.experimental import pallas as pl
from jax.experimental.pallas import tpu as pltpu
from jax.experimental.pallas import tpu_sc as plsc

NC = 2
NS = 16
NW = NC * NS

D = 64
L = 128
P = 65
NBUF = 2


def _emb_body(idx_hbm, table_hbm, out_hbm, idx_v, rows_v, rows_p, trows_v,
              gsem0, gsem1, wsem0, wsem1):
  gsems = (gsem0, gsem1)
  wsems = (wsem0, wsem1)
  wid = lax.axis_index("s") * NC + lax.axis_index("c")
  units = idx_hbm.shape[0] // NW
  row0 = wid * units

  pltpu.sync_copy(idx_hbm.at[pl.ds(row0, units)], idx_v)

  iota16 = lax.iota(jnp.int32, 16)
  rowvecs = [iota16 + 16 * k for k in range(8)]

  def fire_gather(u, b):
    pltpu.async_copy(table_hbm.at[idx_v.at[u]], rows_v.at[b], gsems[b])

  def wait_gather(u, b):
    pltpu.make_async_copy(table_hbm.at[idx_v.at[u]], rows_v.at[b],
                          gsems[b]).wait()

  def transpose(b):
    @plsc.parallel_loop(0, L, unroll=4)
    def _pass1(i):
      for e0 in (0, 16, 32, 48):
        rows_p[b, i, pl.ds(e0, 16)] = rows_v[b, i, pl.ds(e0, 16)]

    @plsc.parallel_loop(0, D, unroll=2)
    def _pass2(e):
      colv = jnp.broadcast_to(e, (16,))
      for k8 in range(8):
        v = plsc.load_gather(rows_p.at[b], [rowvecs[k8], colv])
        trows_v[b, e, pl.ds(16 * k8, 16)] = v

  def fire_wb(u, b):
    q = row0 + u
    s = q // 32
    bt = lax.rem(q, 32)
    for eo in range(8):
      pltpu.async_copy(trows_v.at[b].at[pl.ds(eo * 8, 8)],
                       out_hbm.at[s, eo, bt], wsems[b])

  def wait_wb(u, b):
    q = row0 + u
    s = q // 32
    bt = lax.rem(q, 32)
    for eo in range(8):
      pltpu.make_async_copy(trows_v.at[b].at[pl.ds(eo * 8, 8)],
                            out_hbm.at[s, eo, bt], wsems[b]).wait()

  fire_gather(0, 0)

  wait_gather(0, 0)
  fire_gather(1, 1)
  transpose(0)
  fire_wb(0, 0)

  wait_gather(1, 1)
  fire_gather(2, 0)
  transpose(1)
  fire_wb(1, 1)

  def pair_body(p, carry):
    for db in range(NBUF):
      u = 2 * p + db
      b = db
      nb = 1 - db
      wait_gather(u, b)
      fire_gather(u + 1, nb)
      wait_wb(u - 2, b)
      transpose(b)
      fire_wb(u, b)
    return carry

  lax.fori_loop(1, units // 2 - 1, pair_body, 0)

  wait_gather(units - 2, 0)
  fire_gather(units - 1, 1)
  wait_wb(units - 4, 0)
  transpose(0)
  fire_wb(units - 2, 0)

  wait_gather(units - 1, 1)
  wait_wb(units - 3, 1)
  transpose(1)
  fire_wb(units - 1, 1)

  wait_wb(units - 2, 0)
  wait_wb(units - 1, 1)


def kernel(src, seg, table):
  del seg
  B, S = src.shape
  n = B * S
  idx = src.T.reshape(n // L, L).astype(jnp.int32)
  mesh = plsc.VectorSubcoreMesh(core_axis_name="c", subcore_axis_name="s",
                                num_cores=NC, num_subcores=NS)
  run = pl.kernel(
      _emb_body,
      out_type=jax.ShapeDtypeStruct((S, D // 8, B // L, 8, L), jnp.float32),
      mesh=mesh,
      scratch_types=[
          pltpu.VMEM((n // L // NW, L), jnp.int32),
          pltpu.VMEM((NBUF, L, D), jnp.float32),
          pltpu.VMEM((NBUF, L, P), jnp.float32),
          pltpu.VMEM((NBUF, D, L), jnp.float32),
          pltpu.SemaphoreType.DMA,
          pltpu.SemaphoreType.DMA,
          pltpu.SemaphoreType.DMA,
          pltpu.SemaphoreType.DMA,
      ],
      compiler_params=pltpu.CompilerParams(use_tc_tiling_on_sc=False,
                                           needs_layout_passes=False),
  )
  out5 = run(idx, table)
  return out5.transpose((2, 4, 0, 1, 3)).reshape(B, S, D)


# speedup vs baseline: 2.4241x; 1.0937x over previous
import jax
import jax.numpy as jnp
from jax import lax
from jax.experimental import pallas as pl
from jax.experimental.pallas import tpu as pltpu
from jax.experimental.pallas import tpu_sc as plsc

NC = 2
NS = 16
NW = NC * NS

D = 64
L = 128
P = 65
NBUF = 4


def _emb_body(idx_hbm, table_hbm, out_hbm, idx_v, rows_v, rows_p, trows_v,
              gsem0, gsem1, gsem2, gsem3, wsem0, wsem1, wsem2, wsem3):
  gsems = (gsem0, gsem1, gsem2, gsem3)
  wsems = (wsem0, wsem1, wsem2, wsem3)
  wid = lax.axis_index("s") * NC + lax.axis_index("c")
  units = idx_hbm.shape[0] // NW
  row0 = wid * units

  pltpu.sync_copy(idx_hbm.at[pl.ds(row0, units)], idx_v)

  iota16 = lax.iota(jnp.int32, 16)
  rowvecs = [iota16 + 16 * k for k in range(8)]

  def fire_gather(u, b):
    pltpu.async_copy(table_hbm.at[idx_v.at[u]], rows_v.at[b], gsems[b])

  def wait_gather(u, b):
    pltpu.make_async_copy(table_hbm.at[idx_v.at[u]], rows_v.at[b],
                          gsems[b]).wait()

  def transpose(b):
    @plsc.parallel_loop(0, L, unroll=4)
    def _pass1(i):
      for e0 in (0, 16, 32, 48):
        rows_p[b, i, pl.ds(e0, 16)] = rows_v[b, i, pl.ds(e0, 16)]

    @plsc.parallel_loop(0, D, unroll=2)
    def _pass2(e):
      colv = jnp.broadcast_to(e, (16,))
      for k8 in range(8):
        v = plsc.load_gather(rows_p.at[b], [rowvecs[k8], colv])
        trows_v[b, e, pl.ds(16 * k8, 16)] = v

  def fire_wb(u, b):
    q = row0 + u
    s = q // 32
    bt = lax.rem(q, 32)
    for eo in range(8):
      pltpu.async_copy(trows_v.at[b].at[pl.ds(eo * 8, 8)],
                       out_hbm.at[s, eo, bt], wsems[b])

  def wait_wb(u, b):
    q = row0 + u
    s = q // 32
    bt = lax.rem(q, 32)
    for eo in range(8):
      pltpu.make_async_copy(trows_v.at[b].at[pl.ds(eo * 8, 8)],
                            out_hbm.at[s, eo, bt], wsems[b]).wait()

  for b in range(NBUF):
    fire_gather(b, b)

  for db in range(NBUF):
    wait_gather(db, db)
    transpose(db)
    fire_wb(db, db)
    fire_gather(db + NBUF, db)

  def quad_body(p, carry):
    for db in range(NBUF):
      u = NBUF * p + db
      b = db
      wait_gather(u, b)
      wait_wb(u - NBUF, b)
      transpose(b)
      fire_wb(u, b)
      fire_gather(u + NBUF, b)
    return carry

  lax.fori_loop(1, units // NBUF - 1, quad_body, 0)

  for db in range(NBUF):
    u = units - NBUF + db
    wait_gather(u, db)
    wait_wb(u - NBUF, db)
    transpose(db)
    fire_wb(u, db)

  for db in range(NBUF):
    wait_wb(units - NBUF + db, db)


def kernel(src, seg, table):
  del seg
  B, S = src.shape
  n = B * S
  idx = src.T.reshape(n // L, L).astype(jnp.int32)
  mesh = plsc.VectorSubcoreMesh(core_axis_name="c", subcore_axis_name="s",
                                num_cores=NC, num_subcores=NS)
  run = pl.kernel(
      _emb_body,
      out_type=jax.ShapeDtypeStruct((S, D // 8, B // L, 8, L), jnp.float32),
      mesh=mesh,
      scratch_types=[
          pltpu.VMEM((n // L // NW, L), jnp.int32),
          pltpu.VMEM((NBUF, L, D), jnp.float32),
          pltpu.VMEM((NBUF, L, P), jnp.float32),
          pltpu.VMEM((NBUF, D, L), jnp.float32),
          pltpu.SemaphoreType.DMA,
          pltpu.SemaphoreType.DMA,
          pltpu.SemaphoreType.DMA,
          pltpu.SemaphoreType.DMA,
          pltpu.SemaphoreType.DMA,
          pltpu.SemaphoreType.DMA,
          pltpu.SemaphoreType.DMA,
          pltpu.SemaphoreType.DMA,
      ],
      compiler_params=pltpu.CompilerParams(use_tc_tiling_on_sc=False,
                                           needs_layout_passes=False),
  )
  out5 = run(idx, table)
  return out5.transpose((2, 4, 0, 1, 3)).reshape(B, S, D)
